# Initial kernel scaffold; baseline (speedup 1.0000x reference)
import jax
import jax.numpy as jnp
from jax.experimental import pallas as pl


def kernel(seq1, seq2, adj, sparse, msk, samp_bias1, samp_bias2, W, b, a, Wb, bb):
    raise NotImplementedError("write your pallas kernel here")


# speedup vs baseline: 1.6867x; 1.6867x over previous
import jax
import jax.numpy as jnp
from jax import lax
from jax.experimental import pallas as pl
from jax.experimental.pallas import tpu as pltpu

_BM = 400


def _main_body(adj_ref, seq1_ref, seq2_ref, wt_ref, b_ref, a_ref, msk_ref,
               h_ref, hsum_ref, fts_ref):
    i = pl.program_id(0)
    nh = wt_ref.shape[1]

    @pl.when(i == 0)
    def _init_fts():
        wt = wt_ref[...]
        fts_ref[:, :nh] = jnp.dot(seq1_ref[...], wt,
                                  preferred_element_type=jnp.float32)
        fts_ref[:, nh:] = jnp.dot(seq2_ref[...], wt,
                                  preferred_element_type=jnp.float32)

    h = jnp.dot(adj_ref[...], fts_ref[...],
                preferred_element_type=jnp.float32)
    h = h + b_ref[...]
    h = jnp.where(h >= 0.0, h, a_ref[...] * h)
    h_ref[...] = h
    part = jnp.dot(msk_ref[0], h[:, :nh],
                   preferred_element_type=jnp.float32)

    @pl.when(i == 0)
    def _seed_sum():
        hsum_ref[...] = part

    @pl.when(i > 0)
    def _acc_sum():
        hsum_ref[...] += part


def _score_body(h_ref, hsum_ref, invn_ref, wbt_ref, s_ref):
    nh = wbt_ref.shape[0]
    c = jax.nn.sigmoid(hsum_ref[...] * invn_ref[...])
    v = jnp.dot(c, wbt_ref[...],
                preferred_element_type=jnp.float32)
    z = jnp.zeros_like(v)
    row = lax.broadcasted_iota(jnp.int32, (nh, 2 * nh), 0)
    v1 = jnp.broadcast_to(jnp.concatenate([v, z], axis=1), (nh, 2 * nh))
    v2 = jnp.broadcast_to(jnp.concatenate([z, v], axis=1), (nh, 2 * nh))
    vp = jnp.where(row == 0, v1, 0.0) + jnp.where(row == 1, v2, 0.0)
    dn = (((1,), (1,)), ((), ()))
    s_ref[...] = lax.dot_general(h_ref[...], vp, dn,
                                 preferred_element_type=jnp.float32)


def kernel(seq1, seq2, adj, sparse, msk, samp_bias1, samp_bias2, W, b, a, Wb, bb):
    n = seq1.shape[1]
    nh = W.shape[0]
    adj2 = adj.reshape(n, n)
    s1 = seq1.reshape(n, -1)
    s2 = seq2.reshape(n, -1)
    wt = W.T
    b2 = jnp.concatenate([b, b]).reshape(1, 2 * nh)
    a2 = jnp.broadcast_to(a.reshape(1, 1), (1, 2 * nh))

    grid = n // _BM
    H, hsum = pl.pallas_call(
        _main_body,
        grid=(grid,),
        in_specs=[
            pl.BlockSpec((_BM, n), lambda i: (i, 0)),
            pl.BlockSpec((n, nh), lambda i: (0, 0)),
            pl.BlockSpec((n, nh), lambda i: (0, 0)),
            pl.BlockSpec((nh, nh), lambda i: (0, 0)),
            pl.BlockSpec((1, 2 * nh), lambda i: (0, 0)),
            pl.BlockSpec((1, 2 * nh), lambda i: (0, 0)),
            pl.BlockSpec((1, 1, _BM), lambda i: (i, 0, 0)),
        ],
        out_specs=[
            pl.BlockSpec((_BM, 2 * nh), lambda i: (i, 0)),
            pl.BlockSpec((1, nh), lambda i: (0, 0)),
        ],
        out_shape=[
            jax.ShapeDtypeStruct((n, 2 * nh), jnp.float32),
            jax.ShapeDtypeStruct((1, nh), jnp.float32),
        ],
        scratch_shapes=[pltpu.VMEM((n, 2 * nh), jnp.float32)],
        compiler_params=pltpu.CompilerParams(
            dimension_semantics=("arbitrary",)),
    )(adj2, s1, s2, wt, b2, a2, msk.reshape(grid, 1, _BM))

    invn = jnp.broadcast_to((1.0 / jnp.sum(msk)).reshape(1, 1), (1, nh))
    wbt = Wb[0].T

    S = pl.pallas_call(
        _score_body,
        grid=(1,),
        in_specs=[
            pl.BlockSpec((n, 2 * nh), lambda i: (0, 0)),
            pl.BlockSpec((1, nh), lambda i: (0, 0)),
            pl.BlockSpec((1, nh), lambda i: (0, 0)),
            pl.BlockSpec((nh, nh), lambda i: (0, 0)),
        ],
        out_specs=pl.BlockSpec((n, nh), lambda i: (0, 0)),
        out_shape=jax.ShapeDtypeStruct((n, nh), jnp.float32),
    )(H, hsum, invn, wbt)

    sc1 = S[:, 0].reshape(1, n) + bb + samp_bias1
    sc2 = S[:, 1].reshape(1, n) + bb + samp_bias2
    return jnp.concatenate([sc1, sc2], axis=1)
